# baseline (speedup 1.0000x reference)
import functools

import jax
import jax.numpy as jnp
from jax import lax
from jax.experimental import pallas as pl
from jax.experimental.pallas import tpu as pltpu

try:
    from jax.experimental.pallas import tpu_sc as plsc
except ImportError:
    plsc = None

E = 1600000
N_ATOMS = 100000
ATOM_DIM = 32
PW = ATOM_DIM // 2
BLK = 16000
Q = BLK // 8
C = 1000

_SLOPE = 11.0 / 48.0


def _make_sc_gather():
    info = plsc.get_sparse_core_info()
    nw = info.num_cores * info.num_subcores
    ew = E // nw
    iters = ew // C
    assert ew % C == 0 and C % 8 == 0 and Q % C == 0

    mesh = plsc.VectorSubcoreMesh(core_axis_name="c", subcore_axis_name="s")

    @functools.partial(
        pl.kernel,
        mesh=mesh,
        out_type=(
            jax.ShapeDtypeStruct((E // 8, 8 * PW), jnp.int32),
            jax.ShapeDtypeStruct((E // 8, 8 * PW), jnp.int32),
        ),
        scratch_types=[
            pltpu.VMEM((2, C), jnp.int32),
            pltpu.VMEM((2, C), jnp.int32),
            pltpu.VMEM((2, C, PW), jnp.int32),
            pltpu.VMEM((2, C, PW), jnp.int32),
            pltpu.SemaphoreType.DMA,
            pltpu.SemaphoreType.DMA,
            pltpu.SemaphoreType.DMA,
            pltpu.SemaphoreType.DMA,
        ],
        compiler_params=pltpu.CompilerParams(use_tc_tiling_on_sc=False),
    )
    def gather_kernel(atoms_hbm, idx1_hbm, idx2_hbm, out1_hbm, out2_hbm,
                      idx1_v, idx2_v, rows1_v, rows2_v, *sems):
        wid = lax.axis_index("s") * info.num_cores + lax.axis_index("c")
        ubase = wid * iters
        sem1 = sems[0:2]
        sem2 = sems[2:4]

        def start(i, b):
            off = (ubase + i) * C
            pltpu.sync_copy(idx1_hbm.at[pl.ds(off, C)], idx1_v.at[b])
            pltpu.sync_copy(idx2_hbm.at[pl.ds(off, C)], idx2_v.at[b])
            pltpu.async_copy(atoms_hbm.at[idx1_v.at[b]], rows1_v.at[b], sem1[b])
            pltpu.async_copy(atoms_hbm.at[idx2_v.at[b]], rows2_v.at[b], sem2[b])

        def drain(i, b):
            u = ubase + i
            per_blk = BLK // C
            per_grp = Q // C
            i_blk = u // per_blk
            k = (u % per_blk) // per_grp
            r = (u % per_grp) * C
            row = i_blk * Q + r
            col = PW * k
            pltpu.make_async_copy(
                atoms_hbm.at[idx1_v.at[b]], rows1_v.at[b], sem1[b]).wait()
            pltpu.make_async_copy(
                atoms_hbm.at[idx2_v.at[b]], rows2_v.at[b], sem2[b]).wait()
            pltpu.sync_copy(rows1_v.at[b],
                            out1_hbm.at[pl.ds(row, C), pl.ds(col, PW)])
            pltpu.sync_copy(rows2_v.at[b],
                            out2_hbm.at[pl.ds(row, C), pl.ds(col, PW)])

        start(0, 0)

        def body(j, _):
            for b in range(2):
                i = 2 * j + b

                if b == 0:
                    start(i + 1, 1)
                else:
                    @pl.when(j < iters // 2 - 1)
                    def _():
                        start(i + 1, 0)

                drain(i, b)
            return 0

        lax.fori_loop(0, iters // 2, body, 0)

    return gather_kernel


Q4 = BLK // 4


def _make_sc_unpack():
    info = plsc.get_sparse_core_info()
    nw = info.num_cores * info.num_subcores
    ew = E // nw
    iters = ew // C
    assert Q4 % C == 0

    mesh = plsc.VectorSubcoreMesh(core_axis_name="c", subcore_axis_name="s")

    @functools.partial(
        pl.kernel,
        mesh=mesh,
        out_type=jax.ShapeDtypeStruct((E, 32), jnp.float32),
        scratch_types=[
            pltpu.VMEM((C, 32), jnp.float32),
        ],
        compiler_params=pltpu.CompilerParams(use_tc_tiling_on_sc=False),
    )
    def unpack_kernel(packed_hbm, out_hbm, buf_v):
        wid = lax.axis_index("s") * info.num_cores + lax.axis_index("c")
        ubase = wid * iters

        def body(i, _):
            u = ubase + i
            off = u * C
            per_blk = BLK // C
            per_grp = Q4 // C
            i_blk = u // per_blk
            k = (u % per_blk) // per_grp
            r = (u % per_grp) * C
            row = i_blk * Q4 + r
            col = 32 * k
            pltpu.sync_copy(packed_hbm.at[pl.ds(row, C), pl.ds(col, 32)],
                            buf_v)
            pltpu.sync_copy(buf_v, out_hbm.at[pl.ds(off, C)])
            return 0

        lax.fori_loop(0, iters, body, 0)

    return unpack_kernel


def _unpack_halves(p):
    x = jnp.concatenate(
        [p[:, k * PW:(k + 1) * PW] for k in range(8)], axis=0)
    assert x.shape == (BLK, PW)
    lo = lax.bitcast_convert_type(x << 16, jnp.float32)
    hi = lax.bitcast_convert_type(x & jnp.int32(-65536), jnp.float32)
    return lo, hi


def _mlp_body(a1_ref, a2_ref, b_ref, w1s_ref, b1_ref, w2_ref,
              b2_ref, w3_ref, b3_ref, o_ref):
    dot = functools.partial(jnp.dot, preferred_element_type=jnp.float32)
    lo1, hi1 = _unpack_halves(a1_ref[...])
    lo2, hi2 = _unpack_halves(a2_ref[...])
    w1s = w1s_ref[...]
    h = (dot(lo1, w1s[0:16]) + dot(hi1, w1s[16:32])
         + dot(lo2, w1s[32:48]) + dot(hi2, w1s[48:64])
         + dot(b_ref[...], w1s_ref[64:96, :]) + b1_ref[...])
    h = jnp.where(h >= 0, h, _SLOPE * h)
    h = dot(h, w2_ref[...]) + b2_ref[...]
    h = jnp.where(h >= 0, h, _SLOPE * h)
    h = dot(h, w3_ref[...]) + b3_ref[...]
    o_ref[...] = jnp.concatenate(
        [h[k * Q4:(k + 1) * Q4] for k in range(4)], axis=1)


def _mlp_call(a1, a2, bonds, W1s, b1, W2, b2, W3, b3):
    grid = (E // BLK,)
    full = lambda i: (0, 0)
    row = lambda i: (i, 0)
    return pl.pallas_call(
        _mlp_body,
        grid=grid,
        in_specs=[
            pl.BlockSpec((Q, 8 * PW), row),
            pl.BlockSpec((Q, 8 * PW), row),
            pl.BlockSpec((BLK, ATOM_DIM), row),
            pl.BlockSpec(W1s.shape, full),
            pl.BlockSpec((1, 64), full),
            pl.BlockSpec(W2.shape, full),
            pl.BlockSpec((1, 64), full),
            pl.BlockSpec(W3.shape, full),
            pl.BlockSpec((1, 32), full),
        ],
        out_specs=pl.BlockSpec((BLK // 4, 128), row),
        out_shape=jax.ShapeDtypeStruct((E // 4, 128), jnp.float32),
        compiler_params=pltpu.CompilerParams(
            dimension_semantics=("arbitrary",),
        ),
    )(a1, a2, bonds, W1s, b1, W2, b2, W3, b3)


def kernel(bonds, bond_atom_1, bond_atom_2, atoms, W1, b1, W2, b2, W3, b3):
    atoms_p = lax.bitcast_convert_type(
        atoms.astype(jnp.bfloat16).reshape(N_ATOMS, PW, 2), jnp.int32)
    gather = _make_sc_gather()
    a1, a2 = gather(atoms_p, bond_atom_1.astype(jnp.int32),
                    bond_atom_2.astype(jnp.int32))
    W1a, W1b, W1c = W1[0:32], W1[32:64], W1[64:96]
    W1s = jnp.concatenate(
        [W1a[0::2], W1a[1::2], W1b[0::2], W1b[1::2], W1c], axis=0)
    packed = _mlp_call(a1, a2, bonds, W1s, b1.reshape(1, 64), W2,
                       b2.reshape(1, 64), W3, b3.reshape(1, 32))
    return _make_sc_unpack()(packed)


# speedup vs baseline: 1.2365x; 1.2365x over previous
import functools

import jax
import jax.numpy as jnp
from jax import lax
from jax.experimental import pallas as pl
from jax.experimental.pallas import tpu as pltpu

try:
    from jax.experimental.pallas import tpu_sc as plsc
except ImportError:
    plsc = None

E = 1600000
N_ATOMS = 100000
ATOM_DIM = 32
PW = ATOM_DIM // 2
BLK = 16000
Q = BLK // 8
C = 2000

_SLOPE = 11.0 / 48.0


def _make_sc_gather():
    info = plsc.get_sparse_core_info()
    nw = info.num_cores * info.num_subcores
    ew = E // nw
    iters = ew // C
    assert ew % C == 0 and C % 8 == 0 and Q % C == 0

    mesh = plsc.VectorSubcoreMesh(core_axis_name="c", subcore_axis_name="s")

    @functools.partial(
        pl.kernel,
        mesh=mesh,
        out_type=(
            jax.ShapeDtypeStruct((E // 8, 8 * PW), jnp.int32),
            jax.ShapeDtypeStruct((E // 8, 8 * PW), jnp.int32),
        ),
        scratch_types=[
            pltpu.VMEM((C,), jnp.int32),
            pltpu.VMEM((C,), jnp.int32),
            pltpu.VMEM((C, PW), jnp.int32),
            pltpu.VMEM((C, PW), jnp.int32),
            pltpu.SemaphoreType.DMA,
            pltpu.SemaphoreType.DMA,
        ],
        compiler_params=pltpu.CompilerParams(use_tc_tiling_on_sc=False),
    )
    def gather_kernel(atoms_hbm, idx1_hbm, idx2_hbm, out1_hbm, out2_hbm,
                      idx1_v, idx2_v, rows1_v, rows2_v, sem1, sem2):
        wid = lax.axis_index("s") * info.num_cores + lax.axis_index("c")
        ubase = wid * iters

        def body(i, _):
            u = ubase + i
            off = u * C
            per_blk = BLK // C
            per_grp = Q // C
            i_blk = u // per_blk
            k = (u % per_blk) // per_grp
            r = (u % per_grp) * C
            row = i_blk * Q + r
            col = PW * k
            pltpu.sync_copy(idx1_hbm.at[pl.ds(off, C)], idx1_v)
            pltpu.sync_copy(idx2_hbm.at[pl.ds(off, C)], idx2_v)
            cp1 = pltpu.async_copy(atoms_hbm.at[idx1_v], rows1_v, sem1)
            cp2 = pltpu.async_copy(atoms_hbm.at[idx2_v], rows2_v, sem2)
            cp1.wait()
            cp2.wait()
            pltpu.sync_copy(rows1_v,
                            out1_hbm.at[pl.ds(row, C), pl.ds(col, PW)])
            pltpu.sync_copy(rows2_v,
                            out2_hbm.at[pl.ds(row, C), pl.ds(col, PW)])
            return 0

        lax.fori_loop(0, iters, body, 0)

    return gather_kernel


def _unpack_halves(p):
    x = jnp.concatenate(
        [p[:, k * PW:(k + 1) * PW] for k in range(8)], axis=0)
    assert x.shape == (BLK, PW)
    lo = lax.bitcast_convert_type(x << 16, jnp.float32)
    hi = lax.bitcast_convert_type(x & jnp.int32(-65536), jnp.float32)
    return lo, hi


def _mlp_body(a1_ref, a2_ref, b_ref, w1s_ref, b1_ref, w2_ref,
              b2_ref, w3_ref, b3_ref, o_ref):
    dot = functools.partial(jnp.dot, preferred_element_type=jnp.float32)
    lo1, hi1 = _unpack_halves(a1_ref[...])
    lo2, hi2 = _unpack_halves(a2_ref[...])
    w1s = w1s_ref[...]
    h = (dot(lo1, w1s[0:16]) + dot(hi1, w1s[16:32])
         + dot(lo2, w1s[32:48]) + dot(hi2, w1s[48:64])
         + dot(b_ref[...], w1s_ref[64:96, :]) + b1_ref[...])
    h = jnp.where(h >= 0, h, _SLOPE * h)
    h = dot(h, w2_ref[...]) + b2_ref[...]
    h = jnp.where(h >= 0, h, _SLOPE * h)
    o_ref[...] = dot(h, w3_ref[...]) + b3_ref[...]


def _mlp_call(a1, a2, bonds, W1s, b1, W2, b2, W3, b3):
    grid = (E // BLK,)
    full = lambda i: (0, 0)
    row = lambda i: (i, 0)
    return pl.pallas_call(
        _mlp_body,
        grid=grid,
        in_specs=[
            pl.BlockSpec((Q, 8 * PW), row),
            pl.BlockSpec((Q, 8 * PW), row),
            pl.BlockSpec((BLK, ATOM_DIM), row),
            pl.BlockSpec(W1s.shape, full),
            pl.BlockSpec((1, 64), full),
            pl.BlockSpec(W2.shape, full),
            pl.BlockSpec((1, 64), full),
            pl.BlockSpec(W3.shape, full),
            pl.BlockSpec((1, 32), full),
        ],
        out_specs=pl.BlockSpec((BLK, 32), row),
        out_shape=jax.ShapeDtypeStruct((E, 32), jnp.float32),
        compiler_params=pltpu.CompilerParams(
            dimension_semantics=("arbitrary",),
        ),
    )(a1, a2, bonds, W1s, b1, W2, b2, W3, b3)


def kernel(bonds, bond_atom_1, bond_atom_2, atoms, W1, b1, W2, b2, W3, b3):
    atoms_p = lax.bitcast_convert_type(
        atoms.astype(jnp.bfloat16).reshape(N_ATOMS, PW, 2), jnp.int32)
    gather = _make_sc_gather()
    a1, a2 = gather(atoms_p, bond_atom_1.astype(jnp.int32),
                    bond_atom_2.astype(jnp.int32))
    W1a, W1b, W1c = W1[0:32], W1[32:64], W1[64:96]
    W1s = jnp.concatenate(
        [W1a[0::2], W1a[1::2], W1b[0::2], W1b[1::2], W1c], axis=0)
    return _mlp_call(a1, a2, bonds, W1s, b1.reshape(1, 64), W2,
                     b2.reshape(1, 64), W3, b3.reshape(1, 32))


# speedup vs baseline: 1.2666x; 1.0244x over previous
import functools

import jax
import jax.numpy as jnp
from jax import lax
from jax.experimental import pallas as pl
from jax.experimental.pallas import tpu as pltpu

try:
    from jax.experimental.pallas import tpu_sc as plsc
except ImportError:
    plsc = None

E = 1600000
N_ATOMS = 100000
ATOM_DIM = 32
PW = ATOM_DIM
BLK = 16000
Q = BLK // 4
C = 1000

_SLOPE = 11.0 / 48.0


def _make_sc_gather():
    info = plsc.get_sparse_core_info()
    nw = info.num_cores * info.num_subcores
    ew = E // nw
    iters = ew // C
    assert ew % C == 0 and C % 8 == 0 and Q % C == 0

    mesh = plsc.VectorSubcoreMesh(core_axis_name="c", subcore_axis_name="s")

    @functools.partial(
        pl.kernel,
        mesh=mesh,
        out_type=(
            jax.ShapeDtypeStruct((E // 4, 4 * PW), jnp.float32),
            jax.ShapeDtypeStruct((E // 4, 4 * PW), jnp.float32),
        ),
        scratch_types=[
            pltpu.VMEM((C,), jnp.int32),
            pltpu.VMEM((C,), jnp.int32),
            pltpu.VMEM((C, PW), jnp.float32),
            pltpu.VMEM((C, PW), jnp.float32),
            pltpu.SemaphoreType.DMA,
            pltpu.SemaphoreType.DMA,
        ],
        compiler_params=pltpu.CompilerParams(use_tc_tiling_on_sc=False),
    )
    def gather_kernel(atoms_hbm, idx1_hbm, idx2_hbm, out1_hbm, out2_hbm,
                      idx1_v, idx2_v, rows1_v, rows2_v, sem1, sem2):
        wid = lax.axis_index("s") * info.num_cores + lax.axis_index("c")
        ubase = wid * iters

        def body(i, _):
            u = ubase + i
            off = u * C
            per_blk = BLK // C
            per_grp = Q // C
            i_blk = u // per_blk
            k = (u % per_blk) // per_grp
            r = (u % per_grp) * C
            row = i_blk * Q + r
            col = PW * k
            pltpu.sync_copy(idx1_hbm.at[pl.ds(off, C)], idx1_v)
            pltpu.sync_copy(idx2_hbm.at[pl.ds(off, C)], idx2_v)
            cp1 = pltpu.async_copy(atoms_hbm.at[idx1_v], rows1_v, sem1)
            cp2 = pltpu.async_copy(atoms_hbm.at[idx2_v], rows2_v, sem2)
            cp1.wait()
            cp2.wait()
            pltpu.sync_copy(rows1_v,
                            out1_hbm.at[pl.ds(row, C), pl.ds(col, PW)])
            pltpu.sync_copy(rows2_v,
                            out2_hbm.at[pl.ds(row, C), pl.ds(col, PW)])
            return 0

        lax.fori_loop(0, iters, body, 0)

    return gather_kernel


def _unpack(p):
    x = jnp.concatenate(
        [p[:, k * PW:(k + 1) * PW] for k in range(4)], axis=0)
    assert x.shape == (BLK, PW)
    return x


def _mlp_body(a1_ref, a2_ref, b_ref, w1_ref, b1_ref, w2_ref,
              b2_ref, w3_ref, b3_ref, o_ref):
    dot = functools.partial(jnp.dot, preferred_element_type=jnp.float32)
    h = jnp.concatenate(
        [_unpack(a1_ref[...]), _unpack(a2_ref[...]), b_ref[...]], axis=1)
    h = dot(h, w1_ref[...]) + b1_ref[...]
    h = jnp.where(h >= 0, h, _SLOPE * h)
    h = dot(h, w2_ref[...]) + b2_ref[...]
    h = jnp.where(h >= 0, h, _SLOPE * h)
    o_ref[...] = dot(h, w3_ref[...]) + b3_ref[...]


def _mlp_call(a1, a2, bonds, W1s, b1, W2, b2, W3, b3):
    grid = (E // BLK,)
    full = lambda i: (0, 0)
    row = lambda i: (i, 0)
    return pl.pallas_call(
        _mlp_body,
        grid=grid,
        in_specs=[
            pl.BlockSpec((Q, 4 * PW), row),
            pl.BlockSpec((Q, 4 * PW), row),
            pl.BlockSpec((BLK, ATOM_DIM), row),
            pl.BlockSpec(W1s.shape, full),
            pl.BlockSpec((1, 64), full),
            pl.BlockSpec(W2.shape, full),
            pl.BlockSpec((1, 64), full),
            pl.BlockSpec(W3.shape, full),
            pl.BlockSpec((1, 32), full),
        ],
        out_specs=pl.BlockSpec((BLK, 32), row),
        out_shape=jax.ShapeDtypeStruct((E, 32), jnp.float32),
        compiler_params=pltpu.CompilerParams(
            dimension_semantics=("arbitrary",),
        ),
    )(a1, a2, bonds, W1s, b1, W2, b2, W3, b3)


def kernel(bonds, bond_atom_1, bond_atom_2, atoms, W1, b1, W2, b2, W3, b3):
    gather = _make_sc_gather()
    a1, a2 = gather(atoms, bond_atom_1.astype(jnp.int32),
                    bond_atom_2.astype(jnp.int32))
    return _mlp_call(a1, a2, bonds, W1, b1.reshape(1, 64), W2,
                     b2.reshape(1, 64), W3, b3.reshape(1, 32))


# speedup vs baseline: 1.2747x; 1.0064x over previous
import functools

import jax
import jax.numpy as jnp
from jax import lax
from jax.experimental import pallas as pl
from jax.experimental.pallas import tpu as pltpu

try:
    from jax.experimental.pallas import tpu_sc as plsc
except ImportError:
    plsc = None

E = 1600000
N_ATOMS = 100000
ATOM_DIM = 32
PW = ATOM_DIM // 2
BLK = 16000
Q = BLK // 8
C = 1000

_SLOPE = 11.0 / 48.0


def _make_sc_gather():
    info = plsc.get_sparse_core_info()
    nw = info.num_cores * info.num_subcores
    ew = E // nw
    iters = ew // C
    assert ew % C == 0 and C % 8 == 0 and Q % C == 0

    mesh = plsc.VectorSubcoreMesh(core_axis_name="c", subcore_axis_name="s")

    @functools.partial(
        pl.kernel,
        mesh=mesh,
        out_type=(
            jax.ShapeDtypeStruct((E // 8, 8 * PW), jnp.int32),
            jax.ShapeDtypeStruct((E // 8, 8 * PW), jnp.int32),
        ),
        scratch_types=[
            pltpu.VMEM((C,), jnp.int32),
            pltpu.VMEM((C,), jnp.int32),
            pltpu.VMEM((C, PW), jnp.int32),
            pltpu.VMEM((C, PW), jnp.int32),
            pltpu.SemaphoreType.DMA,
            pltpu.SemaphoreType.DMA,
        ],
        compiler_params=pltpu.CompilerParams(use_tc_tiling_on_sc=False),
    )
    def gather_kernel(atoms_hbm, idx1_hbm, idx2_hbm, out1_hbm, out2_hbm,
                      idx1_v, idx2_v, rows1_v, rows2_v, sem1, sem2):
        wid = lax.axis_index("s") * info.num_cores + lax.axis_index("c")
        ubase = wid * iters

        def body(i, _):
            u = ubase + i
            off = u * C
            per_blk = BLK // C
            per_grp = Q // C
            i_blk = u // per_blk
            k = (u % per_blk) // per_grp
            r = (u % per_grp) * C
            row = i_blk * Q + r
            col = PW * k
            pltpu.sync_copy(idx1_hbm.at[pl.ds(off, C)], idx1_v)
            pltpu.sync_copy(idx2_hbm.at[pl.ds(off, C)], idx2_v)
            cp1 = pltpu.async_copy(atoms_hbm.at[idx1_v], rows1_v, sem1)
            cp2 = pltpu.async_copy(atoms_hbm.at[idx2_v], rows2_v, sem2)
            cp1.wait()
            cp2.wait()
            pltpu.sync_copy(rows1_v,
                            out1_hbm.at[pl.ds(row, C), pl.ds(col, PW)])
            pltpu.sync_copy(rows2_v,
                            out2_hbm.at[pl.ds(row, C), pl.ds(col, PW)])
            return 0

        lax.fori_loop(0, iters, body, 0)

    return gather_kernel


def _round_bf16_bits(x):
    ui = lax.bitcast_convert_type(x, jnp.int32)
    rounded = ui + jnp.int32(0x7FFF) + ((ui >> 16) & jnp.int32(1))
    return lax.shift_right_logical(rounded, jnp.int32(16))


def _pack_table_body(x_ref, o_ref):
    x = x_ref[...]
    lo = _round_bf16_bits(x[:, 0:PW])
    hi = _round_bf16_bits(x[:, PW:2 * PW])
    o_ref[...] = lo | (hi << 16)


def _pack_table(atoms):
    nb = 10000
    return pl.pallas_call(
        _pack_table_body,
        grid=(N_ATOMS // nb,),
        in_specs=[pl.BlockSpec((nb, ATOM_DIM), lambda i: (i, 0))],
        out_specs=pl.BlockSpec((nb, PW), lambda i: (i, 0)),
        out_shape=jax.ShapeDtypeStruct((N_ATOMS, PW), jnp.int32),
    )(atoms)


def _unpack_halves(p):
    x = jnp.concatenate(
        [p[:, k * PW:(k + 1) * PW] for k in range(8)], axis=0)
    assert x.shape == (BLK, PW)
    lo = lax.bitcast_convert_type(x << 16, jnp.float32)
    hi = lax.bitcast_convert_type(x & jnp.int32(-65536), jnp.float32)
    return lo, hi


def _mlp_body(a1_ref, a2_ref, b_ref, w1_ref, b1_ref, w2_ref,
              b2_ref, w3_ref, b3_ref, o_ref):
    dot = functools.partial(jnp.dot, preferred_element_type=jnp.float32)
    lo1, hi1 = _unpack_halves(a1_ref[...])
    lo2, hi2 = _unpack_halves(a2_ref[...])
    w1 = w1_ref[...]
    h = (dot(lo1, w1[0:16]) + dot(hi1, w1[16:32])
         + dot(lo2, w1[32:48]) + dot(hi2, w1[48:64])
         + dot(b_ref[...], w1[64:96]) + b1_ref[...])
    h = jnp.where(h >= 0, h, _SLOPE * h)
    h = dot(h, w2_ref[...]) + b2_ref[...]
    h = jnp.where(h >= 0, h, _SLOPE * h)
    o_ref[...] = dot(h, w3_ref[...]) + b3_ref[...]


def _mlp_call(a1, a2, bonds, W1s, b1, W2, b2, W3, b3):
    grid = (E // BLK,)
    full = lambda i: (0, 0)
    row = lambda i: (i, 0)
    return pl.pallas_call(
        _mlp_body,
        grid=grid,
        in_specs=[
            pl.BlockSpec((Q, 8 * PW), row),
            pl.BlockSpec((Q, 8 * PW), row),
            pl.BlockSpec((BLK, ATOM_DIM), row),
            pl.BlockSpec(W1s.shape, full),
            pl.BlockSpec((1, 64), full),
            pl.BlockSpec(W2.shape, full),
            pl.BlockSpec((1, 64), full),
            pl.BlockSpec(W3.shape, full),
            pl.BlockSpec((1, 32), full),
        ],
        out_specs=pl.BlockSpec((BLK, 32), row),
        out_shape=jax.ShapeDtypeStruct((E, 32), jnp.float32),
        compiler_params=pltpu.CompilerParams(
            dimension_semantics=("arbitrary",),
        ),
    )(a1, a2, bonds, W1s, b1, W2, b2, W3, b3)


def kernel(bonds, bond_atom_1, bond_atom_2, atoms, W1, b1, W2, b2, W3, b3):
    gather = _make_sc_gather()
    a1, a2 = gather(_pack_table(atoms), bond_atom_1.astype(jnp.int32),
                    bond_atom_2.astype(jnp.int32))
    return _mlp_call(a1, a2, bonds, W1, b1.reshape(1, 64), W2,
                     b2.reshape(1, 64), W3, b3.reshape(1, 32))
